# baseline (speedup 1.0000x reference)
import jax
import jax.numpy as jnp
from jax import lax
from jax.experimental import pallas as pl
from jax.experimental.pallas import tpu as pltpu

B, S, D, O = 8, 2048, 1024, 1024
N = B * S
CM = 1024
NCHUNK = N // CM
RING = 6
TM = 512


def _body(x_hbm, w_hbm, b_hbm, m_hbm, o_hbm,
          xbuf, obuf, wb, biasv, maskv,
          in_sems, out_sems, w_sem, b_sem, m_sem):
    pltpu.make_async_copy(w_hbm, xbuf.at[0], w_sem).start()
    pltpu.make_async_copy(b_hbm, biasv, b_sem).start()
    pltpu.make_async_copy(m_hbm, maskv, m_sem).start()
    pltpu.make_async_copy(w_hbm, xbuf.at[0], w_sem).wait()
    wb[...] = xbuf[0].astype(jnp.bfloat16)

    for r in range(RING):
        for k in range(CM // TM):
            pltpu.make_async_copy(
                x_hbm.at[pl.ds(r * CM + k * TM, TM), :],
                xbuf.at[r].at[pl.ds(k * TM, TM), :], in_sems.at[r]
            ).start()

    pltpu.make_async_copy(b_hbm, biasv, b_sem).wait()
    pltpu.make_async_copy(m_hbm, maskv, m_sem).wait()
    bias = biasv[...]

    for i in range(NCHUNK):
        slot = i % RING
        if i >= RING:
            for k in range(CM // TM):
                pltpu.make_async_copy(
                    obuf.at[slot].at[pl.ds(k * TM, TM), :],
                    o_hbm.at[pl.ds((i - RING) * CM + k * TM, TM), :],
                    out_sems.at[slot],
                ).wait()
        for k in range(CM // TM):
            sl = pl.ds(k * TM, TM)
            pltpu.make_async_copy(
                x_hbm.at[pl.ds(i * CM + k * TM, TM), :],
                xbuf.at[slot].at[sl, :], in_sems.at[slot]
            ).wait()
            xb = xbuf[slot, sl, :].astype(jnp.bfloat16)
            acc = lax.dot_general(
                xb, wb[...],
                dimension_numbers=(((1,), (1,)), ((), ())),
                preferred_element_type=jnp.float32,
            )
            mf = maskv[pl.ds(i * CM + k * TM, TM), :]
            obuf[slot, sl, :] = (acc + bias) * mf
            pltpu.make_async_copy(
                obuf.at[slot].at[sl, :],
                o_hbm.at[pl.ds(i * CM + k * TM, TM), :],
                out_sems.at[slot],
            ).start()
        nxt = i + RING
        if nxt < NCHUNK:
            for k in range(CM // TM):
                pltpu.make_async_copy(
                    x_hbm.at[pl.ds(nxt * CM + k * TM, TM), :],
                    xbuf.at[slot].at[pl.ds(k * TM, TM), :],
                    in_sems.at[slot],
                ).start()

    for i in range(NCHUNK - RING, NCHUNK):
        slot = i % RING
        for k in range(CM // TM):
            pltpu.make_async_copy(
                obuf.at[slot].at[pl.ds(k * TM, TM), :],
                o_hbm.at[pl.ds(i * CM + k * TM, TM), :],
                out_sems.at[slot],
            ).wait()


def kernel(input, data_mask, W, b):
    x = input.reshape(N, D)
    m2 = data_mask.reshape(N, 1).astype(jnp.float32)
    b2 = b.reshape(1, O)

    out = pl.pallas_call(
        _body,
        in_specs=[
            pl.BlockSpec(memory_space=pl.ANY),
            pl.BlockSpec(memory_space=pl.ANY),
            pl.BlockSpec(memory_space=pl.ANY),
            pl.BlockSpec(memory_space=pl.ANY),
        ],
        out_specs=pl.BlockSpec(memory_space=pl.ANY),
        out_shape=jax.ShapeDtypeStruct((N, O), jnp.float32),
        scratch_shapes=[
            pltpu.VMEM((RING, CM, D), jnp.float32),
            pltpu.VMEM((RING, CM, O), jnp.float32),
            pltpu.VMEM((O, D), jnp.bfloat16),
            pltpu.VMEM((1, O), jnp.float32),
            pltpu.VMEM((N, 1), jnp.float32),
            pltpu.SemaphoreType.DMA((RING,)),
            pltpu.SemaphoreType.DMA((RING,)),
            pltpu.SemaphoreType.DMA,
            pltpu.SemaphoreType.DMA,
            pltpu.SemaphoreType.DMA,
        ],
        compiler_params=pltpu.CompilerParams(
            vmem_limit_bytes=60 * 1024 * 1024,
        ),
    )(x, W, b2, m2)
    return out.reshape(B, S, O)


# speedup vs baseline: 1.1233x; 1.1233x over previous
import jax
import jax.numpy as jnp
from jax import lax
from jax.experimental import pallas as pl
from jax.experimental.pallas import tpu as pltpu

B, S, D, O = 8, 2048, 1024, 1024
N = B * S
CM = 2048
NCHUNK = N // CM
RIN = 3
ROUT = 2
TM = 512
KSUB = CM // TM


def _body(x_hbm, w_hbm, b_hbm, m_hbm, o_hbm,
          xbuf, obuf, wb, biasv, maskv,
          in_sems, out_sems, w_sem, b_sem, m_sem):
    pltpu.make_async_copy(w_hbm, xbuf.at[0, pl.ds(0, 1024), :], w_sem).start()
    pltpu.make_async_copy(b_hbm, biasv, b_sem).start()
    pltpu.make_async_copy(m_hbm, maskv, m_sem).start()
    pltpu.make_async_copy(w_hbm, xbuf.at[0, pl.ds(0, 1024), :], w_sem).wait()
    wb[...] = xbuf[0, pl.ds(0, 1024), :].astype(jnp.bfloat16)

    for r in range(RIN):
        pltpu.make_async_copy(
            x_hbm.at[pl.ds(r * CM, CM), :], xbuf.at[r], in_sems.at[r]
        ).start()

    pltpu.make_async_copy(b_hbm, biasv, b_sem).wait()
    pltpu.make_async_copy(m_hbm, maskv, m_sem).wait()
    bias = biasv[...]

    for i in range(NCHUNK):
        slot = i % RIN
        oslot = i % ROUT
        pltpu.make_async_copy(
            x_hbm.at[pl.ds(i * CM, CM), :], xbuf.at[slot], in_sems.at[slot]
        ).wait()
        if i >= ROUT:
            for k in range(KSUB):
                pltpu.make_async_copy(
                    obuf.at[oslot].at[pl.ds(k * TM, TM), :],
                    o_hbm.at[pl.ds((i - ROUT) * CM + k * TM, TM), :],
                    out_sems.at[oslot],
                ).wait()
        for k in range(KSUB):
            sl = pl.ds(k * TM, TM)
            xb = xbuf[slot, sl, :].astype(jnp.bfloat16)
            acc = lax.dot_general(
                xb, wb[...],
                dimension_numbers=(((1,), (1,)), ((), ())),
                preferred_element_type=jnp.float32,
            )
            mf = maskv[pl.ds(i * CM + k * TM, TM), :]
            obuf[oslot, sl, :] = (acc + bias) * mf
            pltpu.make_async_copy(
                obuf.at[oslot].at[sl, :],
                o_hbm.at[pl.ds(i * CM + k * TM, TM), :],
                out_sems.at[oslot],
            ).start()
        nxt = i + RIN
        if nxt < NCHUNK:
            pltpu.make_async_copy(
                x_hbm.at[pl.ds(nxt * CM, CM), :], xbuf.at[slot],
                in_sems.at[slot],
            ).start()

    for i in range(NCHUNK - ROUT, NCHUNK):
        oslot = i % ROUT
        for k in range(KSUB):
            pltpu.make_async_copy(
                obuf.at[oslot].at[pl.ds(k * TM, TM), :],
                o_hbm.at[pl.ds(i * CM + k * TM, TM), :],
                out_sems.at[oslot],
            ).wait()


def kernel(input, data_mask, W, b):
    x = input.reshape(N, D)
    m2 = data_mask.reshape(N, 1).astype(jnp.float32)
    b2 = b.reshape(1, O)

    out = pl.pallas_call(
        _body,
        in_specs=[
            pl.BlockSpec(memory_space=pl.ANY),
            pl.BlockSpec(memory_space=pl.ANY),
            pl.BlockSpec(memory_space=pl.ANY),
            pl.BlockSpec(memory_space=pl.ANY),
        ],
        out_specs=pl.BlockSpec(memory_space=pl.ANY),
        out_shape=jax.ShapeDtypeStruct((N, O), jnp.float32),
        scratch_shapes=[
            pltpu.VMEM((RIN, CM, D), jnp.float32),
            pltpu.VMEM((ROUT, CM, O), jnp.float32),
            pltpu.VMEM((O, D), jnp.bfloat16),
            pltpu.VMEM((1, O), jnp.float32),
            pltpu.VMEM((N, 1), jnp.float32),
            pltpu.SemaphoreType.DMA((RIN,)),
            pltpu.SemaphoreType.DMA((ROUT,)),
            pltpu.SemaphoreType.DMA,
            pltpu.SemaphoreType.DMA,
            pltpu.SemaphoreType.DMA,
        ],
        compiler_params=pltpu.CompilerParams(
            vmem_limit_bytes=60 * 1024 * 1024,
        ),
    )(x, W, b2, m2)
    return out.reshape(B, S, O)
